# Initial kernel scaffold; baseline (speedup 1.0000x reference)
import jax
import jax.numpy as jnp
from jax.experimental import pallas as pl


def kernel(x_user, x_item, edge_index_ui, edge_index_iu, Win, b_in, Wk, bk, Wq, bq, Wv, bv, Wo, bo, skip, a_rel, m_rel, p_rel, Wout, bout):
    raise NotImplementedError("write your pallas kernel here")


# speedup vs baseline: 13.8624x; 13.8624x over previous
import functools
import math

import jax
import jax.numpy as jnp
from jax import lax
from jax.experimental import pallas as pl
from jax.experimental.pallas import tpu as pltpu
from jax.experimental.pallas import tpu_sc as plsc

N = 50000
D = 128
H = 4
DH = 32
E = 300000

NTILE = 32
CB = 96
NCH = E // CB
CH_BASE = NCH // NTILE
CH_EXTRA = NCH - CH_BASE * NTILE
MAXCH = CH_BASE + 1
TILE_E = MAXCH * CB
E_PAD = (NTILE - 1) * CH_BASE * CB + 21 * CB + TILE_E
NPAD = 51200
STRIPE = NPAD // 16


def _mm_body(x_ref, w_ref, b_ref, o_ref, *, act):
    y = jnp.dot(x_ref[...], w_ref[...], preferred_element_type=jnp.float32)
    y = y + b_ref[...]
    if act == "relu":
        y = jnp.maximum(y, 0.0)
    o_ref[...] = y


def _mm(x, w, b, act=None, bn=2000):
    n, k = x.shape
    m = w.shape[1]
    grid = (n // bn,)
    return pl.pallas_call(
        functools.partial(_mm_body, act=act),
        grid=grid,
        in_specs=[
            pl.BlockSpec((bn, k), lambda i: (i, 0)),
            pl.BlockSpec((k, m), lambda i: (0, 0)),
            pl.BlockSpec((1, m), lambda i: (0, 0)),
        ],
        out_specs=pl.BlockSpec((bn, m), lambda i: (i, 0)),
        out_shape=jax.ShapeDtypeStruct((n, m), jnp.float32),
    )(x, w, b.reshape(1, m))


def _qkv_body(x_ref, w_ref, b_ref, q_ref, k_ref, v_ref):
    y = jnp.dot(x_ref[...], w_ref[...], preferred_element_type=jnp.float32)
    y = y + b_ref[...]
    q_ref[...] = y[:, 0:D]
    k_ref[...] = y[:, D:2 * D]
    v_ref[...] = y[:, 2 * D:3 * D]


def _qkv(x, wcat, bcat, bn=2000):
    n = x.shape[0]
    grid = (n // bn,)
    sds = jax.ShapeDtypeStruct((n, D), jnp.float32)
    return pl.pallas_call(
        _qkv_body,
        grid=grid,
        in_specs=[
            pl.BlockSpec((bn, D), lambda i: (i, 0)),
            pl.BlockSpec((D, 3 * D), lambda i: (0, 0)),
            pl.BlockSpec((1, 3 * D), lambda i: (0, 0)),
        ],
        out_specs=[pl.BlockSpec((bn, D), lambda i: (i, 0))] * 3,
        out_shape=[sds, sds, sds],
    )(x, wcat, bcat.reshape(1, 3 * D))


def _outstage_body(a0_ref, a1_ref, dn_ref, h_ref, p8_ref, wo_ref, bo_ref,
                   g_ref, o_ref):
    den = jnp.dot(dn_ref[...], p8_ref[...],
                  preferred_element_type=jnp.float32)
    denr = 1.0 / (den + 1e-16)
    a = (a0_ref[...] + a1_ref[...]) * denr
    g = jax.nn.gelu(a)
    o = jnp.dot(g, wo_ref[...], preferred_element_type=jnp.float32)
    o = o + bo_ref[...]
    o_ref[...] = jnp.maximum(o + g_ref[0, 0] * h_ref[...], 0.0)


def _outstage(agg0, agg1, den8, h, p8, wo_b, bo_b, gamma, bn=2000):
    n = h.shape[0]
    grid = (n // bn,)
    return pl.pallas_call(
        _outstage_body,
        grid=grid,
        in_specs=[
            pl.BlockSpec((bn, D), lambda i: (i, 0)),
            pl.BlockSpec((bn, D), lambda i: (i, 0)),
            pl.BlockSpec((bn, 8), lambda i: (i, 0)),
            pl.BlockSpec((bn, D), lambda i: (i, 0)),
            pl.BlockSpec((8, D), lambda i: (0, 0)),
            pl.BlockSpec((D, D), lambda i: (0, 0)),
            pl.BlockSpec((1, D), lambda i: (0, 0)),
            pl.BlockSpec((1, 1), lambda i: (0, 0)),
        ],
        out_specs=pl.BlockSpec((bn, D), lambda i: (i, 0)),
        out_shape=jax.ShapeDtypeStruct((n, D), jnp.float32),
    )(agg0, agg1, den8, h, p8, wo_b, bo_b.reshape(1, D),
      gamma.reshape(1, 1))


def _edge_body(qT, kT, vT, srcI, dstI, zA, zD, aggp, denp,
               ss, s4, d4, d96, qb, kb, vb, exb,
               aggp_s, den_s, sem):
    cid = lax.axis_index("c")
    sid = lax.axis_index("s")
    wid = sid * 2 + cid
    nch = CH_BASE + jnp.where(wid < CH_EXTRA, 1, 0)
    start = CH_BASE * wid + jnp.minimum(wid, CH_EXTRA)

    iota16 = lax.broadcasted_iota(jnp.int32, (16,), 0)

    for h in range(H):
        pltpu.sync_copy(zA, aggp_s.at[pl.ds(sid * STRIPE, STRIPE)])
        pltpu.sync_copy(zD, den_s.at[pl.ds(sid * STRIPE, STRIPE)])
        plsc.subcore_barrier()

        def chunk_body(j, carry):
            off = (start + j) * CB
            c1 = pltpu.async_copy(srcI.at[pl.ds(off, CB)], ss, sem)
            c2 = pltpu.async_copy(dstI.at[pl.ds(off, CB)], d96, sem)
            c1.wait()
            c2.wait()

            def idx_body(g, c):
                sl = pl.ds(g * 16, 16)
                sv = ss[sl]
                dv = d96[sl]
                s4[sl] = sv * 4 + h
                d4[sl] = dv * 4 + h
                return c

            lax.fori_loop(0, CB // 16, idx_body, 0)

            c1 = pltpu.async_copy(qT.at[d4], qb, sem)
            c2 = pltpu.async_copy(kT.at[s4], kb, sem)
            c3 = pltpu.async_copy(vT.at[s4], vb, sem)
            c1.wait()
            c2.wait()
            c3.wait()

            def grp_body(g, c):
                rows = g * 16 + iota16
                acc = jnp.zeros((16,), jnp.float32)
                for jj in range(DH):
                    jv = jnp.full((16,), jj, jnp.int32)
                    acc = acc + (plsc.load_gather(qb, [rows, jv]) *
                                 plsc.load_gather(kb, [rows, jv]))
                ex = jnp.exp(jnp.minimum(acc, 80.0))
                exb[pl.ds(g * 16, 16)] = ex
                for jj in range(DH):
                    jv = jnp.full((16,), jj, jnp.int32)
                    plsc.store_scatter(
                        vb, [rows, jv],
                        plsc.load_gather(vb, [rows, jv]) * ex)
                return c

            lax.fori_loop(0, CB // 16, grp_body, 0)

            pltpu.sync_copy(vb, aggp_s.at[d96], add=True)
            pltpu.sync_copy(exb, den_s.at[d96], add=True)
            return carry

        lax.fori_loop(0, nch, chunk_body, 0)
        plsc.subcore_barrier()

        sl = pl.ds(sid * STRIPE, STRIPE)
        pltpu.sync_copy(aggp_s.at[sl], aggp.at[cid, h, sl])
        pltpu.sync_copy(den_s.at[sl], denp.at[cid, h, sl])
        plsc.subcore_barrier()


def _edge_call(qT, kT, vT, src, dst):
    mesh = plsc.VectorSubcoreMesh(core_axis_name="c", subcore_axis_name="s",
                                  num_cores=2, num_subcores=16)
    fn = pl.kernel(
        _edge_body,
        out_type=(
            jax.ShapeDtypeStruct((2, H, NPAD, DH), jnp.float32),
            jax.ShapeDtypeStruct((2, H, NPAD), jnp.float32),
        ),
        mesh=mesh,
        compiler_params=pltpu.CompilerParams(
            needs_layout_passes=False, use_tc_tiling_on_sc=False),
        scratch_types=[
            pltpu.VMEM((CB,), jnp.int32),
            pltpu.VMEM((CB,), jnp.int32),
            pltpu.VMEM((CB,), jnp.int32),
            pltpu.VMEM((CB,), jnp.int32),
            pltpu.VMEM((CB, DH), jnp.float32),
            pltpu.VMEM((CB, DH), jnp.float32),
            pltpu.VMEM((CB, DH), jnp.float32),
            pltpu.VMEM((CB,), jnp.float32),
            pltpu.VMEM_SHARED((NPAD, DH), jnp.float32),
            pltpu.VMEM_SHARED((NPAD,), jnp.float32),
            pltpu.SemaphoreType.DMA,
        ],
    )
    zA = jnp.zeros((STRIPE, DH), jnp.float32)
    zD = jnp.zeros((STRIPE,), jnp.float32)
    return fn(qT, kT, vT, src, dst, zA, zD)


def kernel(x_user, x_item, edge_index_ui, edge_index_iu, Win, b_in, Wk, bk,
           Wq, bq, Wv, bv, Wo, bo, skip, a_rel, m_rel, p_rel, Wout, bout):
    L = Wk.shape[0]
    scale = p_rel / math.sqrt(DH)

    srcs = [edge_index_ui[0], edge_index_iu[0]]
    dsts = [edge_index_ui[1], edge_index_iu[1]]

    cols = jnp.arange(D) // DH
    j8 = jnp.arange(8) % H
    p8 = (j8[:, None] == cols[None, :]).astype(jnp.float32)

    h = [_mm(x_user, Win[0], b_in[0], act="relu"),
         _mm(x_item, Win[1], b_in[1], act="relu")]

    rels = [(0, 1), (1, 0)]

    for l in range(L):
        qs, kf, vf = {}, {}, {}
        for t in range(2):
            r_d = 1 - t
            r_s = t
            wq_s = (Wq[l, t].reshape(D, H, DH) *
                    scale[l, r_d][None, :, None]).reshape(D, D)
            bq_s = (bq[l, t].reshape(H, DH) *
                    scale[l, r_d][:, None]).reshape(D)
            wk_f = jnp.einsum("dhk,hke->dhe",
                              Wk[l, t].reshape(D, H, DH),
                              a_rel[l, r_s]).reshape(D, D)
            bk_f = jnp.einsum("hk,hke->he", bk[l, t].reshape(H, DH),
                              a_rel[l, r_s]).reshape(D)
            wv_f = jnp.einsum("dhk,hke->dhe",
                              Wv[l, t].reshape(D, H, DH),
                              m_rel[l, r_s]).reshape(D, D)
            bv_f = jnp.einsum("hk,hke->he", bv[l, t].reshape(H, DH),
                              m_rel[l, r_s]).reshape(D)
            wcat = jnp.concatenate([wq_s, wk_f, wv_f], axis=1)
            bcat = jnp.concatenate([bq_s, bk_f, bv_f], axis=0)
            q_t, k_t, v_t = _qkv(h[t], wcat, bcat)
            qs[t], kf[t], vf[t] = q_t, k_t, v_t

        agg8 = {}
        den8 = {}
        for r, (st, dt) in enumerate(rels):
            aggp, denp = _edge_call(
                qs[dt].reshape(4 * N, DH),
                kf[st].reshape(4 * N, DH),
                vf[st].reshape(4 * N, DH),
                srcs[r], dsts[r])
            a = aggp[:, :, :N, :].transpose(0, 2, 1, 3).reshape(2, N, D)
            agg8[dt] = (a[0], a[1])
            den8[dt] = denp[:, :, :N].reshape(8, N).T

        newh = []
        for t in range(2):
            beta = jax.nn.sigmoid(skip[l, t])
            wo_b = Wo[l, t] * beta
            bo_b = bo[l, t] * beta
            gamma = (1.0 - beta).reshape(1, 1)
            a0, a1 = agg8[t]
            newh.append(_outstage(a0, a1, den8[t], h[t], p8, wo_b, bo_b,
                                  gamma))
        h = newh

    out_user = _mm(h[0], Wout[0], bout[0])
    out_item = _mm(h[1], Wout[1], bout[1])
    return jnp.concatenate([out_user, out_item], axis=0)


# speedup vs baseline: 15.0418x; 1.0851x over previous
import functools
import math

import jax
import jax.numpy as jnp
from jax import lax
from jax.experimental import pallas as pl
from jax.experimental.pallas import tpu as pltpu
from jax.experimental.pallas import tpu_sc as plsc

N = 50000
D = 128
H = 4
DH = 32
E = 300000

NTILE = 32
CB = 96
NCH = E // CB
CH_BASE = NCH // NTILE
CH_EXTRA = NCH - CH_BASE * NTILE
MAXCH = CH_BASE + 1
TILE_E = MAXCH * CB
NPAD = 51072
STRIPE = NPAD // 16


def _mm_body(x_ref, w_ref, b_ref, o_ref, *, act):
    y = jnp.dot(x_ref[...], w_ref[...], preferred_element_type=jnp.float32)
    y = y + b_ref[...]
    if act == "relu":
        y = jnp.maximum(y, 0.0)
    o_ref[...] = y


def _mm(x, w, b, act=None, bn=2000):
    n, k = x.shape
    m = w.shape[1]
    grid = (n // bn,)
    return pl.pallas_call(
        functools.partial(_mm_body, act=act),
        grid=grid,
        in_specs=[
            pl.BlockSpec((bn, k), lambda i: (i, 0)),
            pl.BlockSpec((k, m), lambda i: (0, 0)),
            pl.BlockSpec((1, m), lambda i: (0, 0)),
        ],
        out_specs=pl.BlockSpec((bn, m), lambda i: (i, 0)),
        out_shape=jax.ShapeDtypeStruct((n, m), jnp.float32),
    )(x, w, b.reshape(1, m))


def _qkv_body(x_ref, w_ref, b_ref, q_ref, kv_ref):
    y = jnp.dot(x_ref[...], w_ref[...], preferred_element_type=jnp.float32)
    y = y + b_ref[...]
    q_ref[...] = y[:, 0:D]
    parts = []
    for h in range(H):
        parts.append(y[:, D + h * DH:D + (h + 1) * DH])
        parts.append(y[:, 2 * D + h * DH:2 * D + (h + 1) * DH])
    kv_ref[...] = jnp.concatenate(parts, axis=1)


def _qkv(x, wcat, bcat, bn=2000):
    n = x.shape[0]
    grid = (n // bn,)
    return pl.pallas_call(
        _qkv_body,
        grid=grid,
        in_specs=[
            pl.BlockSpec((bn, D), lambda i: (i, 0)),
            pl.BlockSpec((D, 3 * D), lambda i: (0, 0)),
            pl.BlockSpec((1, 3 * D), lambda i: (0, 0)),
        ],
        out_specs=[pl.BlockSpec((bn, D), lambda i: (i, 0)),
                   pl.BlockSpec((bn, 2 * D), lambda i: (i, 0))],
        out_shape=[jax.ShapeDtypeStruct((n, D), jnp.float32),
                   jax.ShapeDtypeStruct((n, 2 * D), jnp.float32)],
    )(x, wcat, bcat.reshape(1, 3 * D))


def _outstage_body(a0_ref, a1_ref, dn_ref, h_ref, p8_ref, wo_ref, bo_ref,
                   g_ref, o_ref):
    den = jnp.dot(dn_ref[...], p8_ref[...],
                  preferred_element_type=jnp.float32)
    denr = 1.0 / (den + 1e-16)
    a = (a0_ref[...] + a1_ref[...]) * denr
    g = jax.nn.gelu(a)
    o = jnp.dot(g, wo_ref[...], preferred_element_type=jnp.float32)
    o = o + bo_ref[...]
    o_ref[...] = jnp.maximum(o + g_ref[0, 0] * h_ref[...], 0.0)


def _outstage(agg0, agg1, den8, h, p8, wo_b, bo_b, gamma, bn=2000):
    n = h.shape[0]
    grid = (n // bn,)
    return pl.pallas_call(
        _outstage_body,
        grid=grid,
        in_specs=[
            pl.BlockSpec((bn, D), lambda i: (i, 0)),
            pl.BlockSpec((bn, D), lambda i: (i, 0)),
            pl.BlockSpec((bn, 8), lambda i: (i, 0)),
            pl.BlockSpec((bn, D), lambda i: (i, 0)),
            pl.BlockSpec((8, D), lambda i: (0, 0)),
            pl.BlockSpec((D, D), lambda i: (0, 0)),
            pl.BlockSpec((1, D), lambda i: (0, 0)),
            pl.BlockSpec((1, 1), lambda i: (0, 0)),
        ],
        out_specs=pl.BlockSpec((bn, D), lambda i: (i, 0)),
        out_shape=jax.ShapeDtypeStruct((n, D), jnp.float32),
    )(agg0, agg1, den8, h, p8, wo_b, bo_b.reshape(1, D),
      gamma.reshape(1, 1))


def _edge_body(qT, kvT, srcI, dstI, zA, zD, aggp, denp,
               ss0, s40, d40, d960, qb0, kvb0, mb0, exb0,
               ss1, s41, d41, d961, qb1, kvb1, mb1, exb1,
               aggp_s, den_s, sem0, sem1):
    cid = lax.axis_index("c")
    sid = lax.axis_index("s")
    wid = sid * 2 + cid
    nch = CH_BASE + jnp.where(wid < CH_EXTRA, 1, 0)
    start = CH_BASE * wid + jnp.minimum(wid, CH_EXTRA)

    iota16 = lax.broadcasted_iota(jnp.int32, (16,), 0)
    bufs = ((ss0, s40, d40, d960, qb0, kvb0, mb0, exb0, sem0),
            (ss1, s41, d41, d961, qb1, kvb1, mb1, exb1, sem1))

    def prep_issue(j, h, buf):
        ss, s4, d4, d96, qb, kvb, mb, exb, sem = buf
        cidx = jnp.minimum(start + j, NCH - 1)
        off = cidx * CB
        a = pltpu.async_copy(srcI.at[pl.ds(off, CB)], ss, sem)
        b = pltpu.async_copy(dstI.at[pl.ds(off, CB)], d96, sem)
        a.wait()
        b.wait()
        is_pad = j >= nch

        def idx_body(g, c):
            sl = pl.ds(g * 16, 16)
            sv = ss[sl]
            dv = d96[sl]
            s4[sl] = sv * 4 + h
            d4[sl] = dv * 4 + h
            d96[sl] = jnp.where(is_pad, N + iota16, dv)
            return c

        lax.fori_loop(0, CB // 16, idx_body, 0)
        g1 = pltpu.async_copy(qT.at[d4], qb, sem)
        g2 = pltpu.async_copy(kvT.at[s4], kvb, sem)
        return g1, g2

    def compute_scatter(dsc, buf):
        ss, s4, d4, d96, qb, kvb, mb, exb, sem = buf
        dsc[0].wait()
        dsc[1].wait()

        def grp_body(g, c):
            rows = g * 16 + iota16
            acc = jnp.zeros((16,), jnp.float32)
            for jj in range(DH):
                jv = jnp.full((16,), jj, jnp.int32)
                acc = acc + (plsc.load_gather(qb, [rows, jv]) *
                             plsc.load_gather(kvb, [rows, jv]))
            ex = jnp.exp(jnp.minimum(acc, 80.0))
            exb[pl.ds(g * 16, 16)] = ex
            for jj in range(DH):
                jv = jnp.full((16,), jj, jnp.int32)
                kv = jnp.full((16,), DH + jj, jnp.int32)
                plsc.store_scatter(
                    mb, [rows, jv],
                    plsc.load_gather(kvb, [rows, kv]) * ex)
            return c

        lax.fori_loop(0, CB // 16, grp_body, 0)
        pltpu.sync_copy(mb, aggp_s.at[d96], add=True)
        pltpu.sync_copy(exb, den_s.at[d96], add=True)

    NPAIR = (MAXCH + 1) // 2

    for h in range(H):
        pltpu.sync_copy(zA, aggp_s.at[pl.ds(sid * STRIPE, STRIPE)])
        pltpu.sync_copy(zD, den_s.at[pl.ds(sid * STRIPE, STRIPE)])
        plsc.subcore_barrier()

        d_a = prep_issue(0, h, bufs[0])

        def pair(jj, carry):
            d_b = prep_issue(2 * jj + 1, h, bufs[1])
            compute_scatter(d_a, bufs[0])
            prep_issue(2 * jj + 2, h, bufs[0])
            compute_scatter(d_b, bufs[1])
            return carry

        lax.fori_loop(0, NPAIR, pair, 0)
        d_a[0].wait()
        d_a[1].wait()
        plsc.subcore_barrier()

        sl = pl.ds(sid * STRIPE, STRIPE)
        pltpu.sync_copy(aggp_s.at[sl], aggp.at[cid, h, sl])
        pltpu.sync_copy(den_s.at[sl], denp.at[cid, h, sl])
        plsc.subcore_barrier()


def _edge_call(q, kv, src, dst):
    qT = q.reshape(4 * N, DH)
    kvT = kv.reshape(4 * N, 2 * DH)
    mesh = plsc.VectorSubcoreMesh(core_axis_name="c", subcore_axis_name="s",
                                  num_cores=2, num_subcores=16)
    fn = pl.kernel(
        _edge_body,
        out_type=(
            jax.ShapeDtypeStruct((2, H, NPAD, DH), jnp.float32),
            jax.ShapeDtypeStruct((2, H, NPAD), jnp.float32),
        ),
        mesh=mesh,
        compiler_params=pltpu.CompilerParams(
            needs_layout_passes=False, use_tc_tiling_on_sc=False),
        scratch_types=(
            [pltpu.VMEM((CB,), jnp.int32)] * 4
            + [pltpu.VMEM((CB, DH), jnp.float32),
               pltpu.VMEM((CB, 2 * DH), jnp.float32),
               pltpu.VMEM((CB, DH), jnp.float32),
               pltpu.VMEM((CB,), jnp.float32)]
            + [pltpu.VMEM((CB,), jnp.int32)] * 4
            + [pltpu.VMEM((CB, DH), jnp.float32),
               pltpu.VMEM((CB, 2 * DH), jnp.float32),
               pltpu.VMEM((CB, DH), jnp.float32),
               pltpu.VMEM((CB,), jnp.float32)]
            + [pltpu.VMEM_SHARED((NPAD, DH), jnp.float32),
               pltpu.VMEM_SHARED((NPAD,), jnp.float32),
               pltpu.SemaphoreType.DMA,
               pltpu.SemaphoreType.DMA]
        ),
    )
    zA = jnp.zeros((STRIPE, DH), jnp.float32)
    zD = jnp.zeros((STRIPE,), jnp.float32)
    return fn(qT, kvT, src, dst, zA, zD)


def kernel(x_user, x_item, edge_index_ui, edge_index_iu, Win, b_in, Wk, bk,
           Wq, bq, Wv, bv, Wo, bo, skip, a_rel, m_rel, p_rel, Wout, bout):
    L = Wk.shape[0]
    scale = p_rel / math.sqrt(DH)

    srcs = [edge_index_ui[0], edge_index_iu[0]]
    dsts = [edge_index_ui[1], edge_index_iu[1]]

    cols = jnp.arange(D) // DH
    j8 = jnp.arange(8) % H
    p8 = (j8[:, None] == cols[None, :]).astype(jnp.float32)

    h = [_mm(x_user, Win[0], b_in[0], act="relu"),
         _mm(x_item, Win[1], b_in[1], act="relu")]

    rels = [(0, 1), (1, 0)]

    for l in range(L):
        qs, kf, vf = {}, {}, {}
        for t in range(2):
            r_d = 1 - t
            r_s = t
            wq_s = (Wq[l, t].reshape(D, H, DH) *
                    scale[l, r_d][None, :, None]).reshape(D, D)
            bq_s = (bq[l, t].reshape(H, DH) *
                    scale[l, r_d][:, None]).reshape(D)
            wk_f = jnp.einsum("dhk,hke->dhe",
                              Wk[l, t].reshape(D, H, DH),
                              a_rel[l, r_s]).reshape(D, D)
            bk_f = jnp.einsum("hk,hke->he", bk[l, t].reshape(H, DH),
                              a_rel[l, r_s]).reshape(D)
            wv_f = jnp.einsum("dhk,hke->dhe",
                              Wv[l, t].reshape(D, H, DH),
                              m_rel[l, r_s]).reshape(D, D)
            bv_f = jnp.einsum("hk,hke->he", bv[l, t].reshape(H, DH),
                              m_rel[l, r_s]).reshape(D)
            wcat = jnp.concatenate([wq_s, wk_f, wv_f], axis=1)
            bcat = jnp.concatenate([bq_s, bk_f, bv_f], axis=0)
            q_t, kv_t = _qkv(h[t], wcat, bcat)
            qs[t], kf[t] = q_t, kv_t

        agg8 = {}
        den8 = {}
        for r, (st, dt) in enumerate(rels):
            aggp, denp = _edge_call(qs[dt], kf[st], srcs[r], dsts[r])
            a = aggp[:, :, :N, :].transpose(0, 2, 1, 3).reshape(2, N, D)
            agg8[dt] = (a[0], a[1])
            den8[dt] = denp[:, :, :N].reshape(8, N).T

        newh = []
        for t in range(2):
            beta = jax.nn.sigmoid(skip[l, t])
            wo_b = Wo[l, t] * beta
            bo_b = bo[l, t] * beta
            gamma = (1.0 - beta).reshape(1, 1)
            a0, a1 = agg8[t]
            newh.append(_outstage(a0, a1, den8[t], h[t], p8, wo_b, bo_b,
                                  gamma))
        h = newh

    out_user = _mm(h[0], Wout[0], bout[0])
    out_item = _mm(h[1], Wout[1], bout[1])
    return jnp.concatenate([out_user, out_item], axis=0)


# speedup vs baseline: 17.6297x; 1.1720x over previous
import functools
import math

import jax
import jax.numpy as jnp
from jax import lax
from jax.experimental import pallas as pl
from jax.experimental.pallas import tpu as pltpu
from jax.experimental.pallas import tpu_sc as plsc

N = 50000
D = 128
H = 4
DH = 32
E = 300000

NTILE = 32
CB = 96
NCH = E // CB
CH_BASE = NCH // NTILE
CH_EXTRA = NCH - CH_BASE * NTILE
MAXCH = CH_BASE + 1
TILE_E = MAXCH * CB
NPAD = 50944
STRIPE = NPAD // 16


def _mm_body(x_ref, w_ref, b_ref, o_ref, *, act):
    y = jnp.dot(x_ref[...], w_ref[...], preferred_element_type=jnp.float32)
    y = y + b_ref[...]
    if act == "relu":
        y = jnp.maximum(y, 0.0)
    o_ref[...] = y


def _mm(x, w, b, act=None, bn=2000):
    n, k = x.shape
    m = w.shape[1]
    grid = (n // bn,)
    return pl.pallas_call(
        functools.partial(_mm_body, act=act),
        grid=grid,
        in_specs=[
            pl.BlockSpec((bn, k), lambda i: (i, 0)),
            pl.BlockSpec((k, m), lambda i: (0, 0)),
            pl.BlockSpec((1, m), lambda i: (0, 0)),
        ],
        out_specs=pl.BlockSpec((bn, m), lambda i: (i, 0)),
        out_shape=jax.ShapeDtypeStruct((n, m), jnp.float32),
    )(x, w, b.reshape(1, m))


def _qkv_body(x_ref, w_ref, b_ref, q_ref, kva_ref, kvb_ref):
    y = jnp.dot(x_ref[...], w_ref[...], preferred_element_type=jnp.float32)
    y = y + b_ref[...]
    q_ref[...] = y[:, 0:D]

    def kvslices(h):
        return [y[:, D + h * DH:D + (h + 1) * DH],
                y[:, 2 * D + h * DH:2 * D + (h + 1) * DH]]

    kva_ref[...] = jnp.concatenate(kvslices(0) + kvslices(1), axis=1)
    kvb_ref[...] = jnp.concatenate(kvslices(2) + kvslices(3), axis=1)


def _qkv(x, wcat, bcat, bn=2000):
    n = x.shape[0]
    grid = (n // bn,)
    sds = jax.ShapeDtypeStruct((n, D), jnp.float32)
    return pl.pallas_call(
        _qkv_body,
        grid=grid,
        in_specs=[
            pl.BlockSpec((bn, D), lambda i: (i, 0)),
            pl.BlockSpec((D, 3 * D), lambda i: (0, 0)),
            pl.BlockSpec((1, 3 * D), lambda i: (0, 0)),
        ],
        out_specs=[pl.BlockSpec((bn, D), lambda i: (i, 0))] * 3,
        out_shape=[sds, sds, sds],
    )(x, wcat, bcat.reshape(1, 3 * D))


def _outstage_body(a0_ref, a1_ref, dn_ref, h_ref, p8_ref, wo_ref, bo_ref,
                   g_ref, o_ref):
    den = jnp.dot(dn_ref[...], p8_ref[...],
                  preferred_element_type=jnp.float32)
    denr = 1.0 / (den + 1e-16)
    a = (a0_ref[0] + a1_ref[0]) * denr
    g = jax.nn.gelu(a)
    o = jnp.dot(g, wo_ref[...], preferred_element_type=jnp.float32)
    o = o + bo_ref[...]
    o_ref[...] = jnp.maximum(o + g_ref[0, 0] * h_ref[...], 0.0)


def _outstage(aggp, den8, h, p8, wo_b, bo_b, gamma, bn=2000):
    n = h.shape[0]
    grid = (n // bn,)
    return pl.pallas_call(
        _outstage_body,
        grid=grid,
        in_specs=[
            pl.BlockSpec((1, bn, D), lambda i: (0, i, 0)),
            pl.BlockSpec((1, bn, D), lambda i: (1, i, 0)),
            pl.BlockSpec((bn, 8), lambda i: (i, 0)),
            pl.BlockSpec((bn, D), lambda i: (i, 0)),
            pl.BlockSpec((8, D), lambda i: (0, 0)),
            pl.BlockSpec((D, D), lambda i: (0, 0)),
            pl.BlockSpec((1, D), lambda i: (0, 0)),
            pl.BlockSpec((1, 1), lambda i: (0, 0)),
        ],
        out_specs=pl.BlockSpec((bn, D), lambda i: (i, 0)),
        out_shape=jax.ShapeDtypeStruct((n, D), jnp.float32),
    )(aggp, aggp, den8, h, p8, wo_b, bo_b.reshape(1, D),
      gamma.reshape(1, 1))


def _edge_body(qT, kvTa, kvTb, epack, zA, zD, aggp, denp,
               sd0, s40, d40, d960, qb0, kvb0, mb0, exb0,
               sd1, s41, d41, d961, qb1, kvb1, mb1, exb1,
               aggp_s, den_s, si0, si1, sg0, sg1, ssc0, ssc1):
    cid = lax.axis_index("c")
    sid = lax.axis_index("s")
    wid = sid * 2 + cid
    nch = CH_BASE + jnp.where(wid < CH_EXTRA, 1, 0)
    start = CH_BASE * wid + jnp.minimum(wid, CH_EXTRA)

    iota16 = lax.broadcasted_iota(jnp.int32, (16,), 0)
    bufs = ((sd0, s40, d40, d960, qb0, kvb0, mb0, exb0, si0, sg0, ssc0),
            (sd1, s41, d41, d961, qb1, kvb1, mb1, exb1, si1, sg1, ssc1))

    def issue_idx(j, buf):
        sd, s4, d4, d96, qb, kvb, mb, exb, si, sg, ssc = buf
        cidx = jnp.minimum(start + j, NCH - 1)
        return pltpu.async_copy(epack.at[cidx], sd, si)

    def issue_scat(buf):
        sd, s4, d4, d96, qb, kvb, mb, exb, si, sg, ssc = buf
        da = pltpu.async_copy(mb, aggp_s.at[d96], ssc, add=True)
        db = pltpu.async_copy(exb, den_s.at[d96], ssc, add=True)
        return da, db

    def prep_issue(j, h, buf):
        sd, s4, d4, d96, qb, kvb, mb, exb, si, sg, ssc = buf
        kvT = kvTa if h < 2 else kvTb
        pltpu.make_async_copy(epack.at[0], sd, si).wait()
        pltpu.make_async_copy(mb, aggp_s.at[d96], ssc).wait()
        pltpu.make_async_copy(exb, den_s.at[d96], ssc).wait()
        is_pad = j >= nch

        def idx_body(g, c):
            sl = pl.ds(g * 16, 16)
            sv = sd[sl]
            dv = sd[pl.ds(CB + g * 16, 16)]
            s4[sl] = sv * 2 + (h % 2)
            d4[sl] = dv * 4 + h
            d96[sl] = jnp.where(is_pad, N + iota16, dv)
            return c

        lax.fori_loop(0, CB // 16, idx_body, 0)
        g1 = pltpu.async_copy(qT.at[d4], qb, sg)
        g2 = pltpu.async_copy(kvT.at[s4], kvb, sg)
        issue_idx(j + 2, buf)
        return g1, g2

    def compute_scatter(dsc, buf):
        sd, s4, d4, d96, qb, kvb, mb, exb, si, sg, ssc = buf
        dsc[0].wait()
        dsc[1].wait()

        def grp_body(g, c):
            rows = g * 16 + iota16
            acc = jnp.zeros((16,), jnp.float32)
            for jj in range(DH):
                jv = jnp.full((16,), jj, jnp.int32)
                acc = acc + (plsc.load_gather(qb, [rows, jv]) *
                             plsc.load_gather(kvb, [rows, jv]))
            ex = jnp.exp(jnp.minimum(acc, 80.0))
            exb[pl.ds(g * 16, 16)] = ex
            for jj in range(DH):
                jv = jnp.full((16,), jj, jnp.int32)
                kv = jnp.full((16,), DH + jj, jnp.int32)
                plsc.store_scatter(
                    mb, [rows, jv],
                    plsc.load_gather(kvb, [rows, kv]) * ex)
            return c

        lax.fori_loop(0, CB // 16, grp_body, 0)
        issue_scat(buf)

    NPAIR = (MAXCH + 1) // 2

    for h in range(H):
        pltpu.sync_copy(zA, aggp_s.at[pl.ds(sid * STRIPE, STRIPE)])
        pltpu.sync_copy(zD, den_s.at[pl.ds(sid * STRIPE, STRIPE)])

        def trash_body(g, c):
            sl = pl.ds(g * 16, 16)
            d960[sl] = N + iota16
            d961[sl] = N + iota16
            return c

        lax.fori_loop(0, CB // 16, trash_body, 0)
        plsc.subcore_barrier()
        issue_scat(bufs[0])
        issue_scat(bufs[1])

        issue_idx(0, bufs[0])
        issue_idx(1, bufs[1])
        d_a = prep_issue(0, h, bufs[0])

        def pair(jj, carry):
            d_b = prep_issue(2 * jj + 1, h, bufs[1])
            compute_scatter(d_a, bufs[0])
            prep_issue(2 * jj + 2, h, bufs[0])
            compute_scatter(d_b, bufs[1])
            return carry

        lax.fori_loop(0, NPAIR, pair, 0)
        d_a[0].wait()
        d_a[1].wait()
        pltpu.make_async_copy(epack.at[0], sd0, si0).wait()
        pltpu.make_async_copy(epack.at[0], sd1, si1).wait()
        pltpu.make_async_copy(mb1, aggp_s.at[d961], ssc1).wait()
        pltpu.make_async_copy(exb1, den_s.at[d961], ssc1).wait()
        plsc.subcore_barrier()

        sl = pl.ds(sid * STRIPE, STRIPE)
        pltpu.sync_copy(aggp_s.at[sl], aggp.at[cid, sl, pl.ds(h * DH, DH)])
        pltpu.sync_copy(den_s.at[sl], denp.at[cid, h, sl])
        plsc.subcore_barrier()


def _edge_call(q, kva, kvb, epack):
    qT = q.reshape(4 * N, DH)
    kvTa = kva.reshape(2 * N, 2 * DH)
    kvTb = kvb.reshape(2 * N, 2 * DH)
    mesh = plsc.VectorSubcoreMesh(core_axis_name="c", subcore_axis_name="s",
                                  num_cores=2, num_subcores=16)
    par = [pltpu.VMEM((2 * CB,), jnp.int32),
           pltpu.VMEM((CB,), jnp.int32),
           pltpu.VMEM((CB,), jnp.int32),
           pltpu.VMEM((CB,), jnp.int32),
           pltpu.VMEM((CB, DH), jnp.float32),
           pltpu.VMEM((CB, 2 * DH), jnp.float32),
           pltpu.VMEM((CB, DH), jnp.float32),
           pltpu.VMEM((CB,), jnp.float32)]
    fn = pl.kernel(
        _edge_body,
        out_type=(
            jax.ShapeDtypeStruct((2, NPAD, D), jnp.float32),
            jax.ShapeDtypeStruct((2, H, NPAD), jnp.float32),
        ),
        mesh=mesh,
        compiler_params=pltpu.CompilerParams(
            needs_layout_passes=False, use_tc_tiling_on_sc=False),
        scratch_types=(
            par + par
            + [pltpu.VMEM_SHARED((NPAD, DH), jnp.float32),
               pltpu.VMEM_SHARED((NPAD,), jnp.float32)]
            + [pltpu.SemaphoreType.DMA] * 6
        ),
    )
    zA = jnp.zeros((STRIPE, DH), jnp.float32)
    zD = jnp.zeros((STRIPE,), jnp.float32)
    return fn(qT, kvTa, kvTb, epack, zA, zD)


def kernel(x_user, x_item, edge_index_ui, edge_index_iu, Win, b_in, Wk, bk,
           Wq, bq, Wv, bv, Wo, bo, skip, a_rel, m_rel, p_rel, Wout, bout):
    L = Wk.shape[0]
    scale = p_rel / math.sqrt(DH)

    epacks = [
        jnp.concatenate([ei[0].reshape(NCH, CB), ei[1].reshape(NCH, CB)],
                        axis=1)
        for ei in (edge_index_ui, edge_index_iu)
    ]

    cols = jnp.arange(D) // DH
    j8 = jnp.arange(8) % H
    p8 = (j8[:, None] == cols[None, :]).astype(jnp.float32)

    h = [_mm(x_user, Win[0], b_in[0], act="relu"),
         _mm(x_item, Win[1], b_in[1], act="relu")]

    rels = [(0, 1), (1, 0)]

    for l in range(L):
        qs, kf, vf = {}, {}, {}
        for t in range(2):
            r_d = 1 - t
            r_s = t
            wq_s = (Wq[l, t].reshape(D, H, DH) *
                    scale[l, r_d][None, :, None]).reshape(D, D)
            bq_s = (bq[l, t].reshape(H, DH) *
                    scale[l, r_d][:, None]).reshape(D)
            wk_f = jnp.einsum("dhk,hke->dhe",
                              Wk[l, t].reshape(D, H, DH),
                              a_rel[l, r_s]).reshape(D, D)
            bk_f = jnp.einsum("hk,hke->he", bk[l, t].reshape(H, DH),
                              a_rel[l, r_s]).reshape(D)
            wv_f = jnp.einsum("dhk,hke->dhe",
                              Wv[l, t].reshape(D, H, DH),
                              m_rel[l, r_s]).reshape(D, D)
            bv_f = jnp.einsum("hk,hke->he", bv[l, t].reshape(H, DH),
                              m_rel[l, r_s]).reshape(D)
            wcat = jnp.concatenate([wq_s, wk_f, wv_f], axis=1)
            bcat = jnp.concatenate([bq_s, bk_f, bv_f], axis=0)
            q_t, kva_t, kvb_t = _qkv(h[t], wcat, bcat)
            qs[t], kf[t] = q_t, (kva_t, kvb_t)

        aggs = {}
        den8 = {}
        for r, (st, dt) in enumerate(rels):
            aggp, denp = _edge_call(qs[dt], kf[st][0], kf[st][1], epacks[r])
            aggs[dt] = aggp
            den8[dt] = denp.reshape(8, NPAD).T

        newh = []
        for t in range(2):
            beta = jax.nn.sigmoid(skip[l, t])
            wo_b = Wo[l, t] * beta
            bo_b = bo[l, t] * beta
            gamma = (1.0 - beta).reshape(1, 1)
            newh.append(_outstage(aggs[t], den8[t], h[t], p8, wo_b, bo_b,
                                  gamma))
        h = newh

    out_user = _mm(h[0], Wout[0], bout[0])
    out_item = _mm(h[1], Wout[1], bout[1])
    return jnp.concatenate([out_user, out_item], axis=0)


# speedup vs baseline: 61.7087x; 3.5003x over previous
import functools
import math

import jax
import jax.numpy as jnp
from jax import lax
from jax.experimental import pallas as pl
from jax.experimental.pallas import tpu as pltpu
from jax.experimental.pallas import tpu_sc as plsc

N = 50000
D = 128
H = 4
DH = 32
E = 300000

NTILE = 32
CB = 96
NCH = E // CB
CH_BASE = NCH // NTILE
CH_EXTRA = NCH - CH_BASE * NTILE
MAXCH = CH_BASE + 1
TILE_E = MAXCH * CB
NPAD = 50944
STRIPE = NPAD // 16


def _mm_body(x_ref, w_ref, b_ref, o_ref, *, act):
    y = jnp.dot(x_ref[...], w_ref[...], preferred_element_type=jnp.float32)
    y = y + b_ref[...]
    if act == "relu":
        y = jnp.maximum(y, 0.0)
    o_ref[...] = y


def _mm(x, w, b, act=None, bn=2000):
    n, k = x.shape
    m = w.shape[1]
    grid = (n // bn,)
    return pl.pallas_call(
        functools.partial(_mm_body, act=act),
        grid=grid,
        in_specs=[
            pl.BlockSpec((bn, k), lambda i: (i, 0)),
            pl.BlockSpec((k, m), lambda i: (0, 0)),
            pl.BlockSpec((1, m), lambda i: (0, 0)),
        ],
        out_specs=pl.BlockSpec((bn, m), lambda i: (i, 0)),
        out_shape=jax.ShapeDtypeStruct((n, m), jnp.float32),
    )(x, w, b.reshape(1, m))


def _qkv_body(x_ref, w_ref, b_ref, q_ref, kva_ref, kvb_ref):
    y = jnp.dot(x_ref[...], w_ref[...], preferred_element_type=jnp.float32)
    y = y + b_ref[...]
    q_ref[...] = y[:, 0:D]

    def kvslices(h):
        return [y[:, D + h * DH:D + (h + 1) * DH],
                y[:, 2 * D + h * DH:2 * D + (h + 1) * DH]]

    kva_ref[...] = jnp.concatenate(kvslices(0) + kvslices(1), axis=1)
    kvb_ref[...] = jnp.concatenate(kvslices(2) + kvslices(3), axis=1)


def _qkv(x, wcat, bcat, bn=2000):
    n = x.shape[0]
    grid = (n // bn,)
    sds = jax.ShapeDtypeStruct((n, D), jnp.float32)
    return pl.pallas_call(
        _qkv_body,
        grid=grid,
        in_specs=[
            pl.BlockSpec((bn, D), lambda i: (i, 0)),
            pl.BlockSpec((D, 3 * D), lambda i: (0, 0)),
            pl.BlockSpec((1, 3 * D), lambda i: (0, 0)),
        ],
        out_specs=[pl.BlockSpec((bn, D), lambda i: (i, 0))] * 3,
        out_shape=[sds, sds, sds],
    )(x, wcat, bcat.reshape(1, 3 * D))


def _outstage_body(a0_ref, a1_ref, dn_ref, h_ref, p8_ref, wo_ref, bo_ref,
                   g_ref, o_ref):
    den = jnp.dot(dn_ref[...], p8_ref[...],
                  preferred_element_type=jnp.float32)
    denr = 1.0 / (den + 1e-16)
    a = (a0_ref[0] + a1_ref[0]) * denr
    g = jax.nn.gelu(a)
    o = jnp.dot(g, wo_ref[...], preferred_element_type=jnp.float32)
    o = o + bo_ref[...]
    o_ref[...] = jnp.maximum(o + g_ref[0, 0] * h_ref[...], 0.0)


def _outstage(aggp, den8, h, p8, wo_b, bo_b, gamma, bn=2000):
    n = h.shape[0]
    grid = (n // bn,)
    return pl.pallas_call(
        _outstage_body,
        grid=grid,
        in_specs=[
            pl.BlockSpec((1, bn, D), lambda i: (0, i, 0)),
            pl.BlockSpec((1, bn, D), lambda i: (1, i, 0)),
            pl.BlockSpec((bn, 8), lambda i: (i, 0)),
            pl.BlockSpec((bn, D), lambda i: (i, 0)),
            pl.BlockSpec((8, D), lambda i: (0, 0)),
            pl.BlockSpec((D, D), lambda i: (0, 0)),
            pl.BlockSpec((1, D), lambda i: (0, 0)),
            pl.BlockSpec((1, 1), lambda i: (0, 0)),
        ],
        out_specs=pl.BlockSpec((bn, D), lambda i: (i, 0)),
        out_shape=jax.ShapeDtypeStruct((n, D), jnp.float32),
    )(aggp, aggp, den8, h, p8, wo_b, bo_b.reshape(1, D),
      gamma.reshape(1, 1))


def _edge_body(qT, kvTa, kvTb, epack, zA, zD, aggp, denp,
               sd0, s40, d40, d960, qb0, kvb0, mb0, exb0,
               sd1, s41, d41, d961, qb1, kvb1, mb1, exb1,
               aggp_s, den_s, si0, si1, sg0, sg1, ssc0, ssc1):
    cid = lax.axis_index("c")
    sid = lax.axis_index("s")
    wid = sid * 2 + cid
    nch = CH_BASE + jnp.where(wid < CH_EXTRA, 1, 0)
    start = CH_BASE * wid + jnp.minimum(wid, CH_EXTRA)

    iota16 = lax.broadcasted_iota(jnp.int32, (16,), 0)
    bufs = ((sd0, s40, d40, d960, qb0, kvb0, mb0, exb0, si0, sg0, ssc0),
            (sd1, s41, d41, d961, qb1, kvb1, mb1, exb1, si1, sg1, ssc1))

    def issue_idx(j, buf):
        sd, s4, d4, d96, qb, kvb, mb, exb, si, sg, ssc = buf
        cidx = jnp.minimum(start + j, NCH - 1)
        return pltpu.async_copy(epack.at[cidx], sd, si)


    def issue_scat(buf):
        sd, s4, d4, d96, qb, kvb, mb, exb, si, sg, ssc = buf
        da = pltpu.async_copy(mb, aggp_s.at[d96], ssc, add=True)
        db = pltpu.async_copy(exb, den_s.at[d96], ssc, add=True)
        return da, db

    def prep_issue(j, h, buf):
        sd, s4, d4, d96, qb, kvb, mb, exb, si, sg, ssc = buf
        kvT = kvTa if h < 2 else kvTb
        pltpu.make_async_copy(epack.at[0], sd, si).wait()
        pltpu.make_async_copy(mb, aggp_s.at[d96], ssc).wait()
        pltpu.make_async_copy(exb, den_s.at[d96], ssc).wait()
        is_pad = j >= nch

        def idx_body(g, c):
            sl = pl.ds(g * 16, 16)
            sv = sd[sl]
            dv = sd[pl.ds(CB + g * 16, 16)]
            s4[sl] = sv * 2 + (h % 2)
            d4[sl] = dv * 4 + h
            d96[sl] = jnp.where(is_pad, N + iota16, dv)
            return c

        lax.fori_loop(0, CB // 16, idx_body, 0)
        g1 = pltpu.async_copy(qT.at[d4], qb, sg)
        g2 = pltpu.async_copy(kvT.at[s4], kvb, sg)
        issue_idx(j + 2, buf)
        return g1, g2

    def compute_scatter(dsc, buf):
        sd, s4, d4, d96, qb, kvb, mb, exb, si, sg, ssc = buf
        dsc[0].wait()
        dsc[1].wait()

        def grp_body(g, c):
            av = jnp.zeros((16,), jnp.float32)
            for i in range(16):
                e = g * 16 + i
                q0 = qb[e, pl.ds(0, 16)]
                q1 = qb[e, pl.ds(16, 16)]
                k0 = kvb[e, pl.ds(0, 16)]
                k1 = kvb[e, pl.ds(16, 16)]
                alpha = jnp.sum(q0 * k0 + q1 * k1)
                av = jnp.where(iota16 == i, alpha, av)
            sl = pl.ds(g * 16, 16)
            exv = jnp.exp(jnp.minimum(av, 80.0))
            exb[sl] = exv
            for i in range(16):
                e = g * 16 + i
                ex = exv[i]
                mb[e, pl.ds(0, 16)] = kvb[e, pl.ds(32, 16)] * ex
                mb[e, pl.ds(16, 16)] = kvb[e, pl.ds(48, 16)] * ex
            return c

        lax.fori_loop(0, CB // 16, grp_body, 0)
        issue_scat(buf)

    NPAIR = (MAXCH + 1) // 2

    for h in range(H):
        pltpu.sync_copy(zA, aggp_s.at[pl.ds(sid * STRIPE, STRIPE)])
        pltpu.sync_copy(zD, den_s.at[pl.ds(sid * STRIPE, STRIPE)])

        def trash_body(g, c):
            sl = pl.ds(g * 16, 16)
            d960[sl] = N + iota16
            d961[sl] = N + iota16
            return c

        lax.fori_loop(0, CB // 16, trash_body, 0)
        plsc.subcore_barrier()
        issue_scat(bufs[0])
        issue_scat(bufs[1])

        issue_idx(0, bufs[0])
        issue_idx(1, bufs[1])
        d_a = prep_issue(0, h, bufs[0])

        def pair(jj, carry):
            d_b = prep_issue(2 * jj + 1, h, bufs[1])
            compute_scatter(d_a, bufs[0])
            prep_issue(2 * jj + 2, h, bufs[0])
            compute_scatter(d_b, bufs[1])
            return carry

        lax.fori_loop(0, NPAIR, pair, 0)
        d_a[0].wait()
        d_a[1].wait()
        pltpu.make_async_copy(epack.at[0], sd0, si0).wait()
        pltpu.make_async_copy(epack.at[0], sd1, si1).wait()
        pltpu.make_async_copy(mb1, aggp_s.at[d961], ssc1).wait()
        pltpu.make_async_copy(exb1, den_s.at[d961], ssc1).wait()
        plsc.subcore_barrier()

        sl = pl.ds(sid * STRIPE, STRIPE)
        pltpu.sync_copy(aggp_s.at[sl], aggp.at[cid, sl, pl.ds(h * DH, DH)])
        pltpu.sync_copy(den_s.at[sl], denp.at[cid, h, sl])
        plsc.subcore_barrier()


def _edge_call(q, kva, kvb, epack):
    qT = q.reshape(4 * N, DH)
    kvTa = kva.reshape(2 * N, 2 * DH)
    kvTb = kvb.reshape(2 * N, 2 * DH)
    mesh = plsc.VectorSubcoreMesh(core_axis_name="c", subcore_axis_name="s",
                                  num_cores=2, num_subcores=16)
    par = [pltpu.VMEM((2 * CB,), jnp.int32),
           pltpu.VMEM((CB,), jnp.int32),
           pltpu.VMEM((CB,), jnp.int32),
           pltpu.VMEM((CB,), jnp.int32),
           pltpu.VMEM((CB, DH), jnp.float32),
           pltpu.VMEM((CB, 2 * DH), jnp.float32),
           pltpu.VMEM((CB, DH), jnp.float32),
           pltpu.VMEM((CB,), jnp.float32)]
    fn = pl.kernel(
        _edge_body,
        out_type=(
            jax.ShapeDtypeStruct((2, NPAD, D), jnp.float32),
            jax.ShapeDtypeStruct((2, H, NPAD), jnp.float32),
        ),
        mesh=mesh,
        compiler_params=pltpu.CompilerParams(
            needs_layout_passes=False, use_tc_tiling_on_sc=False),
        scratch_types=(
            par + par
            + [pltpu.VMEM_SHARED((NPAD, DH), jnp.float32),
               pltpu.VMEM_SHARED((NPAD,), jnp.float32)]
            + [pltpu.SemaphoreType.DMA] * 6
        ),
    )
    zA = jnp.zeros((STRIPE, DH), jnp.float32)
    zD = jnp.zeros((STRIPE,), jnp.float32)
    return fn(qT, kvTa, kvTb, epack, zA, zD)


def kernel(x_user, x_item, edge_index_ui, edge_index_iu, Win, b_in, Wk, bk,
           Wq, bq, Wv, bv, Wo, bo, skip, a_rel, m_rel, p_rel, Wout, bout):
    L = Wk.shape[0]
    scale = p_rel / math.sqrt(DH)

    epacks = [
        jnp.concatenate([ei[0].reshape(NCH, CB), ei[1].reshape(NCH, CB)],
                        axis=1)
        for ei in (edge_index_ui, edge_index_iu)
    ]

    cols = jnp.arange(D) // DH
    j8 = jnp.arange(8) % H
    p8 = (j8[:, None] == cols[None, :]).astype(jnp.float32)

    h = [_mm(x_user, Win[0], b_in[0], act="relu"),
         _mm(x_item, Win[1], b_in[1], act="relu")]

    rels = [(0, 1), (1, 0)]

    for l in range(L):
        qs, kf, vf = {}, {}, {}
        for t in range(2):
            r_d = 1 - t
            r_s = t
            wq_s = (Wq[l, t].reshape(D, H, DH) *
                    scale[l, r_d][None, :, None]).reshape(D, D)
            bq_s = (bq[l, t].reshape(H, DH) *
                    scale[l, r_d][:, None]).reshape(D)
            wk_f = jnp.einsum("dhk,hke->dhe",
                              Wk[l, t].reshape(D, H, DH),
                              a_rel[l, r_s]).reshape(D, D)
            bk_f = jnp.einsum("hk,hke->he", bk[l, t].reshape(H, DH),
                              a_rel[l, r_s]).reshape(D)
            wv_f = jnp.einsum("dhk,hke->dhe",
                              Wv[l, t].reshape(D, H, DH),
                              m_rel[l, r_s]).reshape(D, D)
            bv_f = jnp.einsum("hk,hke->he", bv[l, t].reshape(H, DH),
                              m_rel[l, r_s]).reshape(D)
            wcat = jnp.concatenate([wq_s, wk_f, wv_f], axis=1)
            bcat = jnp.concatenate([bq_s, bk_f, bv_f], axis=0)
            q_t, kva_t, kvb_t = _qkv(h[t], wcat, bcat)
            qs[t], kf[t] = q_t, (kva_t, kvb_t)

        aggs = {}
        den8 = {}
        for r, (st, dt) in enumerate(rels):
            aggp, denp = _edge_call(qs[dt], kf[st][0], kf[st][1], epacks[r])
            aggs[dt] = aggp
            den8[dt] = denp.reshape(8, NPAD).T

        newh = []
        for t in range(2):
            beta = jax.nn.sigmoid(skip[l, t])
            wo_b = Wo[l, t] * beta
            bo_b = bo[l, t] * beta
            gamma = (1.0 - beta).reshape(1, 1)
            newh.append(_outstage(aggs[t], den8[t], h[t], p8, wo_b, bo_b,
                                  gamma))
        h = newh

    out_user = _mm(h[0], Wout[0], bout[0])
    out_item = _mm(h[1], Wout[1], bout[1])
    return jnp.concatenate([out_user, out_item], axis=0)
